# baseline (speedup 1.0000x reference)
import functools

import jax
import jax.numpy as jnp
from jax import lax
from jax.experimental import pallas as pl
from jax.experimental.pallas import tpu as pltpu
from jax.experimental.pallas import tpu_sc as plsc

_NC = 2
_NS = 16
_CH = 80


def _sc_degree(E, NPAD):
    epw = E // (_NC * _NS)
    nch = epw // _CH
    rpt = NPAD // _NS
    mesh = plsc.VectorSubcoreMesh(core_axis_name="c", subcore_axis_name="s")

    @functools.partial(
        pl.kernel, mesh=mesh,
        out_type=jax.ShapeDtypeStruct((_NC, NPAD), jnp.float32),
        scratch_types=[
            pltpu.VMEM((nch, _CH), jnp.int32),
            pltpu.VMEM((_CH,), jnp.float32),
            pltpu.VMEM((rpt,), jnp.float32),
            pltpu.VMEM_SHARED((NPAD,), jnp.float32),
        ],
    )
    def k(dst_hbm, out_hbm, didx_v, ones_v, zero_v, acc_sh):
        c = lax.axis_index("c")
        s = lax.axis_index("s")
        w = c * _NS + s

        for i in range(_CH // 16):
            ones_v[pl.ds(i * 16, 16)] = jnp.ones((16,), jnp.float32)

        def zb(i, _):
            zero_v[pl.ds(i * 16, 16)] = jnp.zeros((16,), jnp.float32)
            return 0
        lax.fori_loop(0, rpt // 16, zb, 0)

        pltpu.sync_copy(zero_v, acc_sh.at[pl.ds(s * rpt, rpt)])
        pltpu.sync_copy(dst_hbm.at[w], didx_v)
        plsc.subcore_barrier()

        def body(j, _):
            pltpu.sync_copy(ones_v, acc_sh.at[didx_v.at[j]], add=True)
            return 0
        lax.fori_loop(0, nch, body, 0)

        plsc.subcore_barrier()
        pltpu.sync_copy(acc_sh.at[pl.ds(s * rpt, rpt)],
                        out_hbm.at[c, pl.ds(s * rpt, rpt)])

    return k


def _sc_aggregate(E, NPAD, DH):
    epw = E // _NS
    nch = epw // _CH
    rpt = NPAD // _NS
    nzc = rpt // _CH
    mesh = plsc.VectorSubcoreMesh(core_axis_name="c", subcore_axis_name="s")

    @functools.partial(
        pl.kernel, mesh=mesh,
        compiler_params=pltpu.CompilerParams(use_tc_tiling_on_sc=False),
        out_type=jax.ShapeDtypeStruct((_NC, NPAD, DH), jnp.float32),
        scratch_types=[
            pltpu.VMEM((nch, _CH), jnp.int32),
            pltpu.VMEM((nch, _CH), jnp.int32),
            pltpu.VMEM((_CH, DH), jnp.float32),
            pltpu.VMEM((_CH, DH), jnp.float32),
            pltpu.VMEM((_CH, DH), jnp.float32),
            pltpu.VMEM_SHARED((NPAD, DH), jnp.float32),
            pltpu.SemaphoreType.DMA,
            pltpu.SemaphoreType.DMA,
            pltpu.SemaphoreType.DMA,
            pltpu.SemaphoreType.DMA,
        ],
    )
    def k(src_hbm, dst_hbm, g_hbm, out_hbm, sidx_v, didx_v, rows0_v, rows1_v,
          zrow_v, acc_sh, sem0, sem1, ssem0, ssem1):
        c = lax.axis_index("c")
        s = lax.axis_index("s")

        def zb(r, _):
            for i in range(DH // 16):
                zrow_v[r, pl.ds(i * 16, 16)] = jnp.zeros((16,), jnp.float32)
            return 0
        lax.fori_loop(0, _CH, zb, 0)

        for i in range(nzc):
            pltpu.sync_copy(
                zrow_v, acc_sh.at[pl.ds(s * rpt + i * _CH, _CH)])

        pltpu.sync_copy(src_hbm.at[s], sidx_v)
        pltpu.sync_copy(dst_hbm.at[s], didx_v)
        plsc.subcore_barrier()

        gsrc = g_hbm.at[c]
        pltpu.async_copy(gsrc.at[sidx_v.at[0]], rows0_v, sem0)
        pltpu.async_copy(gsrc.at[sidx_v.at[1]], rows1_v, sem1)

        def body(i, _):
            j0 = 2 * i
            j1 = j0 + 1
            pltpu.make_async_copy(gsrc.at[sidx_v.at[j0]], rows0_v, sem0).wait()
            pltpu.async_copy(rows0_v, acc_sh.at[didx_v.at[j0]], ssem0,
                             add=True)
            pltpu.make_async_copy(gsrc.at[sidx_v.at[j1]], rows1_v, sem1).wait()
            pltpu.async_copy(rows1_v, acc_sh.at[didx_v.at[j1]], ssem1,
                             add=True)

            pltpu.make_async_copy(rows0_v, acc_sh.at[didx_v.at[j0]],
                                  ssem0).wait()

            @pl.when(j0 + 2 < nch)
            def _():
                pltpu.async_copy(gsrc.at[sidx_v.at[j0 + 2]], rows0_v, sem0)

            pltpu.make_async_copy(rows1_v, acc_sh.at[didx_v.at[j1]],
                                  ssem1).wait()

            @pl.when(j1 + 2 < nch)
            def _():
                pltpu.async_copy(gsrc.at[sidx_v.at[j1 + 2]], rows1_v, sem1)
            return 0
        lax.fori_loop(0, nch // 2, body, 0)

        plsc.subcore_barrier()
        for i in range(nzc):
            r0 = s * rpt + i * _CH
            pltpu.sync_copy(acc_sh.at[pl.ds(r0, _CH)],
                            out_hbm.at[c, pl.ds(r0, _CH)])

    return k


def _tc_lin_in(N, D, C):
    DH = D // 2

    def body(x_ref, w_ref, b_ref, dp_ref, o_ref):
        h = jnp.dot(x_ref[...], w_ref[...],
                    preferred_element_type=jnp.float32) + b_ref[...]
        dinv = lax.rsqrt(dp_ref[0] + dp_ref[1] + 1.0)
        g = h * dinv
        o_ref[0] = g[:, :DH]
        o_ref[1] = g[:, DH:]

    return pl.pallas_call(
        body,
        grid=(N // C,),
        in_specs=[
            pl.BlockSpec((C, D), lambda i: (i, 0)),
            pl.BlockSpec((D, D), lambda i: (0, 0)),
            pl.BlockSpec((1, D), lambda i: (0, 0)),
            pl.BlockSpec((2, C, 1), lambda i: (0, i, 0)),
        ],
        out_specs=pl.BlockSpec((2, C, DH), lambda i: (0, i, 0)),
        out_shape=jax.ShapeDtypeStruct((2, N, DH), jnp.float32),
    )


def _tc_gcn_mid(N, D, C):
    DH = D // 2

    def body(p_ref, g_ref, dp_ref, w_ref, b_ref, o_ref):
        dinv = lax.rsqrt(dp_ref[0] + dp_ref[1] + 1.0)
        sg = jnp.concatenate([p_ref[0] + g_ref[0], p_ref[1] + g_ref[1]],
                             axis=1)
        a = sg * dinv
        h = jnp.dot(a, w_ref[...], preferred_element_type=jnp.float32)
        h = jnp.maximum(h + b_ref[...], 0.0)
        g = h * dinv
        o_ref[0] = g[:, :DH]
        o_ref[1] = g[:, DH:]

    return pl.pallas_call(
        body,
        grid=(N // C,),
        in_specs=[
            pl.BlockSpec((2, C, DH), lambda i: (0, i, 0)),
            pl.BlockSpec((2, C, DH), lambda i: (0, i, 0)),
            pl.BlockSpec((2, C, 1), lambda i: (0, i, 0)),
            pl.BlockSpec((D, D), lambda i: (0, 0)),
            pl.BlockSpec((1, D), lambda i: (0, 0)),
        ],
        out_specs=pl.BlockSpec((2, C, DH), lambda i: (0, i, 0)),
        out_shape=jax.ShapeDtypeStruct((2, N, DH), jnp.float32),
    )


def _tc_gcn_pool_head(N, D, C, G):
    nblk = N // C
    DH = D // 2

    def body(p_ref, g_ref, dp_ref, w_ref, b_ref, bt_ref, wo_ref, bo_ref,
             o_ref, pooled, cnt):
        i = pl.program_id(0)
        dinv = lax.rsqrt(dp_ref[0] + dp_ref[1] + 1.0)
        sg = jnp.concatenate([p_ref[0] + g_ref[0], p_ref[1] + g_ref[1]],
                             axis=1)
        a = sg * dinv
        h = jnp.dot(a, w_ref[...], preferred_element_type=jnp.float32)
        h = jnp.maximum(h + b_ref[...], 0.0)

        iota = lax.broadcasted_iota(jnp.int32, (C, G), 1)
        maskT = (bt_ref[...] == iota).astype(jnp.float32)
        pblk = lax.dot_general(maskT, h, (((0,), (0,)), ((), ())),
                               preferred_element_type=jnp.float32)
        ones = jnp.ones((C, 1), jnp.float32)
        cblk = lax.dot_general(maskT, ones, (((0,), (0,)), ((), ())),
                               preferred_element_type=jnp.float32)

        @pl.when(i == 0)
        def _():
            pooled[...] = pblk
            cnt[...] = cblk

        @pl.when(i > 0)
        def _():
            pooled[...] += pblk
            cnt[...] += cblk

        @pl.when(i == nblk - 1)
        def _():
            mean = pooled[...] / jnp.maximum(cnt[...], 1.0)
            o_ref[...] = jnp.dot(mean, wo_ref[...],
                                 preferred_element_type=jnp.float32) + bo_ref[...]

    return pl.pallas_call(
        body,
        grid=(nblk,),
        in_specs=[
            pl.BlockSpec((2, C, DH), lambda i: (0, i, 0)),
            pl.BlockSpec((2, C, DH), lambda i: (0, i, 0)),
            pl.BlockSpec((2, C, 1), lambda i: (0, i, 0)),
            pl.BlockSpec((D, D), lambda i: (0, 0)),
            pl.BlockSpec((1, D), lambda i: (0, 0)),
            pl.BlockSpec((C, 1), lambda i: (i, 0)),
            pl.BlockSpec((D, 1), lambda i: (0, 0)),
            pl.BlockSpec((1, 1), lambda i: (0, 0)),
        ],
        out_specs=pl.BlockSpec((G, 1), lambda i: (0, 0)),
        out_shape=jax.ShapeDtypeStruct((G, 1), jnp.float32),
        scratch_shapes=[
            pltpu.VMEM((G, D), jnp.float32),
            pltpu.VMEM((G, 1), jnp.float32),
        ],
    )


def kernel(x, edge_index, batch, W_in, b_in, W1, b1, W2, b2, W_out, b_out):
    N, D = x.shape
    E = edge_index.shape[1]
    G = 64
    C = 1000
    DH = D // 2
    NPAD = -(-N // (_NS * _CH)) * (_NS * _CH)

    nch_deg = E // (_NC * _NS) // _CH
    dst_deg = edge_index[1].reshape(_NC * _NS, nch_deg, _CH)
    nch = E // _NS // _CH
    src3 = edge_index[0].reshape(_NS, nch, _CH)
    dst3 = edge_index[1].reshape(_NS, nch, _CH)

    degp = _sc_degree(E, NPAD)(dst_deg)
    dp = degp[:, :, None]

    g0 = _tc_lin_in(N, D, C)(x, W_in, b_in.reshape(1, D), dp)
    S = _sc_aggregate(E, NPAD, DH)(src3, dst3, g0)
    g1 = _tc_gcn_mid(N, D, C)(S, g0, dp, W1, b1.reshape(1, D))
    S = _sc_aggregate(E, NPAD, DH)(src3, dst3, g1)
    out = _tc_gcn_pool_head(N, D, C, G)(
        S, g1, dp, W2, b2.reshape(1, D), batch[:, None],
        W_out, b_out.reshape(1, 1))
    return out


# speedup vs baseline: 1.3262x; 1.3262x over previous
import functools

import jax
import jax.numpy as jnp
from jax import lax
from jax.experimental import pallas as pl
from jax.experimental.pallas import tpu as pltpu
from jax.experimental.pallas import tpu_sc as plsc

_NC = 2
_NS = 16
_CH = 128


def _sc_degree(nch, NPAD):
    rpt = NPAD // _NS
    mesh = plsc.VectorSubcoreMesh(core_axis_name="c", subcore_axis_name="s")

    @functools.partial(
        pl.kernel, mesh=mesh,
        out_type=jax.ShapeDtypeStruct((_NC, NPAD), jnp.float32),
        scratch_types=[
            pltpu.VMEM((nch, _CH), jnp.int32),
            pltpu.VMEM((_CH,), jnp.float32),
            pltpu.VMEM((rpt,), jnp.float32),
            pltpu.VMEM_SHARED((NPAD,), jnp.float32),
        ],
    )
    def k(dst_hbm, out_hbm, didx_v, ones_v, zero_v, acc_sh):
        c = lax.axis_index("c")
        s = lax.axis_index("s")
        w = c * _NS + s

        for i in range(_CH // 16):
            ones_v[pl.ds(i * 16, 16)] = jnp.ones((16,), jnp.float32)

        def zb(i, _):
            zero_v[pl.ds(i * 16, 16)] = jnp.zeros((16,), jnp.float32)
            return 0
        lax.fori_loop(0, rpt // 16, zb, 0)

        pltpu.sync_copy(zero_v, acc_sh.at[pl.ds(s * rpt, rpt)])
        pltpu.sync_copy(dst_hbm.at[w], didx_v)
        plsc.subcore_barrier()

        def body(j, _):
            pltpu.sync_copy(ones_v, acc_sh.at[didx_v.at[j]], add=True)
            return 0
        lax.fori_loop(0, nch, body, 0)

        plsc.subcore_barrier()
        pltpu.sync_copy(acc_sh.at[pl.ds(s * rpt, rpt)],
                        out_hbm.at[c, pl.ds(s * rpt, rpt)])

    return k


def _sc_aggregate(nch, NPAD, DH):
    rpt = NPAD // _NS
    nzc = rpt // _CH
    mesh = plsc.VectorSubcoreMesh(core_axis_name="c", subcore_axis_name="s")

    @functools.partial(
        pl.kernel, mesh=mesh,
        compiler_params=pltpu.CompilerParams(use_tc_tiling_on_sc=False),
        out_type=jax.ShapeDtypeStruct((_NC, NPAD, DH), jnp.float32),
        scratch_types=[
            pltpu.VMEM((nch, _CH), jnp.int32),
            pltpu.VMEM((nch, _CH), jnp.int32),
            pltpu.VMEM((_CH, DH), jnp.float32),
            pltpu.VMEM((_CH, DH), jnp.float32),
            pltpu.VMEM((_CH, DH), jnp.float32),
            pltpu.VMEM_SHARED((NPAD, DH), jnp.float32),
            pltpu.SemaphoreType.DMA,
            pltpu.SemaphoreType.DMA,
        ],
    )
    def k(src_hbm, dst_hbm, g_hbm, out_hbm, sidx_v, didx_v, rows0_v, rows1_v,
          zrow_v, acc_sh, sem0, sem1):
        c = lax.axis_index("c")
        s = lax.axis_index("s")

        def zb(r, _):
            for i in range(DH // 16):
                zrow_v[r, pl.ds(i * 16, 16)] = jnp.zeros((16,), jnp.float32)
            return 0
        lax.fori_loop(0, _CH, zb, 0)

        for i in range(nzc):
            pltpu.sync_copy(
                zrow_v, acc_sh.at[pl.ds(s * rpt + i * _CH, _CH)])

        pltpu.sync_copy(src_hbm.at[s], sidx_v)
        pltpu.sync_copy(dst_hbm.at[s], didx_v)
        plsc.subcore_barrier()

        gsrc = g_hbm.at[c]
        pltpu.async_copy(gsrc.at[sidx_v.at[0]], rows0_v, sem0)
        pltpu.async_copy(gsrc.at[sidx_v.at[1]], rows1_v, sem1)

        def body(i, _):
            j0 = 2 * i
            j1 = j0 + 1
            pltpu.make_async_copy(gsrc.at[sidx_v.at[j0]], rows0_v, sem0).wait()
            pltpu.sync_copy(rows0_v, acc_sh.at[didx_v.at[j0]], add=True)

            @pl.when(j0 + 2 < nch)
            def _():
                pltpu.async_copy(gsrc.at[sidx_v.at[j0 + 2]], rows0_v, sem0)

            pltpu.make_async_copy(gsrc.at[sidx_v.at[j1]], rows1_v, sem1).wait()
            pltpu.sync_copy(rows1_v, acc_sh.at[didx_v.at[j1]], add=True)

            @pl.when(j1 + 2 < nch)
            def _():
                pltpu.async_copy(gsrc.at[sidx_v.at[j1 + 2]], rows1_v, sem1)
            return 0
        lax.fori_loop(0, nch // 2, body, 0)

        plsc.subcore_barrier()
        for i in range(nzc):
            r0 = s * rpt + i * _CH
            pltpu.sync_copy(acc_sh.at[pl.ds(r0, _CH)],
                            out_hbm.at[c, pl.ds(r0, _CH)])

    return k


def _tc_lin_in(N, D, C):
    DH = D // 2

    def body(x_ref, w_ref, b_ref, dp_ref, o_ref):
        h = jnp.dot(x_ref[...], w_ref[...],
                    preferred_element_type=jnp.float32) + b_ref[...]
        dinv = lax.rsqrt(dp_ref[0] + dp_ref[1] + 1.0)
        g = h * dinv
        o_ref[0] = g[:, :DH]
        o_ref[1] = g[:, DH:]

    return pl.pallas_call(
        body,
        grid=(N // C,),
        in_specs=[
            pl.BlockSpec((C, D), lambda i: (i, 0)),
            pl.BlockSpec((D, D), lambda i: (0, 0)),
            pl.BlockSpec((1, D), lambda i: (0, 0)),
            pl.BlockSpec((2, C, 1), lambda i: (0, i, 0)),
        ],
        out_specs=pl.BlockSpec((2, C, DH), lambda i: (0, i, 0)),
        out_shape=jax.ShapeDtypeStruct((2, N, DH), jnp.float32),
    )


def _tc_gcn_mid(N, D, C):
    DH = D // 2

    def body(p_ref, g_ref, dp_ref, w_ref, b_ref, o_ref):
        dinv = lax.rsqrt(dp_ref[0] + dp_ref[1] + 1.0)
        sg = jnp.concatenate([p_ref[0] + g_ref[0], p_ref[1] + g_ref[1]],
                             axis=1)
        a = sg * dinv
        h = jnp.dot(a, w_ref[...], preferred_element_type=jnp.float32)
        h = jnp.maximum(h + b_ref[...], 0.0)
        g = h * dinv
        o_ref[0] = g[:, :DH]
        o_ref[1] = g[:, DH:]

    return pl.pallas_call(
        body,
        grid=(N // C,),
        in_specs=[
            pl.BlockSpec((2, C, DH), lambda i: (0, i, 0)),
            pl.BlockSpec((2, C, DH), lambda i: (0, i, 0)),
            pl.BlockSpec((2, C, 1), lambda i: (0, i, 0)),
            pl.BlockSpec((D, D), lambda i: (0, 0)),
            pl.BlockSpec((1, D), lambda i: (0, 0)),
        ],
        out_specs=pl.BlockSpec((2, C, DH), lambda i: (0, i, 0)),
        out_shape=jax.ShapeDtypeStruct((2, N, DH), jnp.float32),
    )


def _tc_gcn_pool_head(N, D, C, G):
    nblk = N // C
    DH = D // 2

    def body(p_ref, g_ref, dp_ref, w_ref, b_ref, bt_ref, wo_ref, bo_ref,
             o_ref, pooled, cnt):
        i = pl.program_id(0)
        dinv = lax.rsqrt(dp_ref[0] + dp_ref[1] + 1.0)
        sg = jnp.concatenate([p_ref[0] + g_ref[0], p_ref[1] + g_ref[1]],
                             axis=1)
        a = sg * dinv
        h = jnp.dot(a, w_ref[...], preferred_element_type=jnp.float32)
        h = jnp.maximum(h + b_ref[...], 0.0)

        iota = lax.broadcasted_iota(jnp.int32, (C, G), 1)
        maskT = (bt_ref[...] == iota).astype(jnp.float32)
        pblk = lax.dot_general(maskT, h, (((0,), (0,)), ((), ())),
                               preferred_element_type=jnp.float32)
        ones = jnp.ones((C, 1), jnp.float32)
        cblk = lax.dot_general(maskT, ones, (((0,), (0,)), ((), ())),
                               preferred_element_type=jnp.float32)

        @pl.when(i == 0)
        def _():
            pooled[...] = pblk
            cnt[...] = cblk

        @pl.when(i > 0)
        def _():
            pooled[...] += pblk
            cnt[...] += cblk

        @pl.when(i == nblk - 1)
        def _():
            mean = pooled[...] / jnp.maximum(cnt[...], 1.0)
            o_ref[...] = jnp.dot(mean, wo_ref[...],
                                 preferred_element_type=jnp.float32) + bo_ref[...]

    return pl.pallas_call(
        body,
        grid=(nblk,),
        in_specs=[
            pl.BlockSpec((2, C, DH), lambda i: (0, i, 0)),
            pl.BlockSpec((2, C, DH), lambda i: (0, i, 0)),
            pl.BlockSpec((2, C, 1), lambda i: (0, i, 0)),
            pl.BlockSpec((D, D), lambda i: (0, 0)),
            pl.BlockSpec((1, D), lambda i: (0, 0)),
            pl.BlockSpec((C, 1), lambda i: (i, 0)),
            pl.BlockSpec((D, 1), lambda i: (0, 0)),
            pl.BlockSpec((1, 1), lambda i: (0, 0)),
        ],
        out_specs=pl.BlockSpec((G, 1), lambda i: (0, 0)),
        out_shape=jax.ShapeDtypeStruct((G, 1), jnp.float32),
        scratch_shapes=[
            pltpu.VMEM((G, D), jnp.float32),
            pltpu.VMEM((G, 1), jnp.float32),
        ],
    )


def kernel(x, edge_index, batch, W_in, b_in, W1, b1, W2, b2, W_out, b_out):
    N, D = x.shape
    E = edge_index.shape[1]
    G = 64
    C = 1000
    DH = D // 2
    NPAD = -(-(N + 1) // (_NS * _CH)) * (_NS * _CH)

    nch = -(-E // (_NS * _CH))
    nch += nch % 2
    padn = _NS * nch * _CH - E
    src = edge_index[0]
    dst = edge_index[1]
    if padn:
        fill = jnp.arange(padn, dtype=jnp.int32)
        src = jnp.concatenate([src, fill % N])
        dst = jnp.concatenate([dst, N + fill % (NPAD - N)])
    src3 = src.reshape(_NS, nch, _CH)
    dst3 = dst.reshape(_NS, nch, _CH)
    nch_deg = nch // 2
    dst_deg = dst.reshape(_NC * _NS, nch_deg, _CH)

    degp = _sc_degree(nch_deg, NPAD)(dst_deg)
    dp = degp[:, :, None]

    g0 = _tc_lin_in(N, D, C)(x, W_in, b_in.reshape(1, D), dp)
    S = _sc_aggregate(nch, NPAD, DH)(src3, dst3, g0)
    g1 = _tc_gcn_mid(N, D, C)(S, g0, dp, W1, b1.reshape(1, D))
    S = _sc_aggregate(nch, NPAD, DH)(src3, dst3, g1)
    out = _tc_gcn_pool_head(N, D, C, G)(
        S, g1, dp, W2, b2.reshape(1, D), batch[:, None],
        W_out, b_out.reshape(1, 1))
    return out


# speedup vs baseline: 1.5869x; 1.1966x over previous
import functools

import jax
import jax.numpy as jnp
from jax import lax
from jax.experimental import pallas as pl
from jax.experimental.pallas import tpu as pltpu
from jax.experimental.pallas import tpu_sc as plsc

_NC = 2
_NS = 16
_CH = 128


def _sc_degree(nch, NPAD):
    rpt = NPAD // _NS
    mesh = plsc.VectorSubcoreMesh(core_axis_name="c", subcore_axis_name="s")

    @functools.partial(
        pl.kernel, mesh=mesh,
        out_type=jax.ShapeDtypeStruct((_NC, NPAD), jnp.float32),
        scratch_types=[
            pltpu.VMEM((nch, _CH), jnp.int32),
            pltpu.VMEM((_CH,), jnp.float32),
            pltpu.VMEM((rpt,), jnp.float32),
            pltpu.VMEM_SHARED((NPAD,), jnp.float32),
        ],
    )
    def k(dst_hbm, out_hbm, didx_v, ones_v, zero_v, acc_sh):
        c = lax.axis_index("c")
        s = lax.axis_index("s")
        w = c * _NS + s

        for i in range(_CH // 16):
            ones_v[pl.ds(i * 16, 16)] = jnp.ones((16,), jnp.float32)

        def zb(i, _):
            zero_v[pl.ds(i * 16, 16)] = jnp.zeros((16,), jnp.float32)
            return 0
        lax.fori_loop(0, rpt // 16, zb, 0)

        pltpu.sync_copy(zero_v, acc_sh.at[pl.ds(s * rpt, rpt)])
        pltpu.sync_copy(dst_hbm.at[w], didx_v)
        plsc.subcore_barrier()

        def body(j, _):
            pltpu.sync_copy(ones_v, acc_sh.at[didx_v.at[j]], add=True)
            return 0
        lax.fori_loop(0, nch, body, 0)

        plsc.subcore_barrier()
        pltpu.sync_copy(acc_sh.at[pl.ds(s * rpt, rpt)],
                        out_hbm.at[c, pl.ds(s * rpt, rpt)])

    return k


def _sc_aggregate(nch, NPAD, DH):
    rpt = NPAD // _NS
    nzc = rpt // _CH
    mesh = plsc.VectorSubcoreMesh(core_axis_name="c", subcore_axis_name="s")

    @functools.partial(
        pl.kernel, mesh=mesh,
        compiler_params=pltpu.CompilerParams(use_tc_tiling_on_sc=False),
        out_type=jax.ShapeDtypeStruct((_NC, NPAD, DH), jnp.float32),
        scratch_types=[
            pltpu.VMEM((nch, _CH), jnp.int32),
            pltpu.VMEM((nch, _CH), jnp.int32),
            pltpu.VMEM((_CH, DH), jnp.float32),
            pltpu.VMEM((_CH, DH), jnp.float32),
            pltpu.VMEM((_CH, DH), jnp.float32),
            pltpu.VMEM((_CH, DH), jnp.float32),
            pltpu.VMEM((_CH, DH), jnp.float32),
            pltpu.VMEM_SHARED((NPAD, DH), jnp.float32),
            pltpu.SemaphoreType.DMA,
            pltpu.SemaphoreType.DMA,
            pltpu.SemaphoreType.DMA,
            pltpu.SemaphoreType.DMA,
        ],
    )
    def k(src_hbm, dst_hbm, g_hbm, out_hbm, sidx_v, didx_v, rows0_v, rows1_v,
          rows2_v, rows3_v, zrow_v, acc_sh, sem0, sem1, sem2, sem3):
        c = lax.axis_index("c")
        s = lax.axis_index("s")

        def zb(r, _):
            for i in range(DH // 16):
                zrow_v[r, pl.ds(i * 16, 16)] = jnp.zeros((16,), jnp.float32)
            return 0
        lax.fori_loop(0, _CH, zb, 0)

        for i in range(nzc):
            pltpu.sync_copy(
                zrow_v, acc_sh.at[pl.ds(s * rpt + i * _CH, _CH)])

        pltpu.sync_copy(src_hbm.at[s], sidx_v)
        pltpu.sync_copy(dst_hbm.at[s], didx_v)
        plsc.subcore_barrier()

        gsrc = g_hbm.at[c]
        bufs = ((rows0_v, sem0), (rows1_v, sem1), (rows2_v, sem2),
                (rows3_v, sem3))
        for b, (rv, sm) in enumerate(bufs):
            pltpu.async_copy(gsrc.at[sidx_v.at[b]], rv, sm)

        def body(i, _):
            for b, (rv, sm) in enumerate(bufs):
                j = 4 * i + b
                pltpu.make_async_copy(gsrc.at[sidx_v.at[j]], rv, sm).wait()
                pltpu.sync_copy(rv, acc_sh.at[didx_v.at[j]], add=True)

                @pl.when(j + 4 < nch)
                def _():
                    pltpu.async_copy(gsrc.at[sidx_v.at[j + 4]], rv, sm)
            return 0
        lax.fori_loop(0, nch // 4, body, 0)

        plsc.subcore_barrier()
        for i in range(nzc):
            r0 = s * rpt + i * _CH
            pltpu.sync_copy(acc_sh.at[pl.ds(r0, _CH)],
                            out_hbm.at[c, pl.ds(r0, _CH)])

    return k


def _tc_lin_in(N, D, C):
    DH = D // 2

    def body(x_ref, w_ref, b_ref, dp_ref, o_ref):
        h = jnp.dot(x_ref[...], w_ref[...],
                    preferred_element_type=jnp.float32) + b_ref[...]
        dinv = lax.rsqrt(dp_ref[0] + dp_ref[1] + 1.0)
        g = h * dinv
        o_ref[0] = g[:, :DH]
        o_ref[1] = g[:, DH:]

    return pl.pallas_call(
        body,
        grid=(N // C,),
        in_specs=[
            pl.BlockSpec((C, D), lambda i: (i, 0)),
            pl.BlockSpec((D, D), lambda i: (0, 0)),
            pl.BlockSpec((1, D), lambda i: (0, 0)),
            pl.BlockSpec((2, C, 1), lambda i: (0, i, 0)),
        ],
        out_specs=pl.BlockSpec((2, C, DH), lambda i: (0, i, 0)),
        out_shape=jax.ShapeDtypeStruct((2, N, DH), jnp.float32),
    )


def _tc_gcn_mid(N, D, C):
    DH = D // 2

    def body(p_ref, g_ref, dp_ref, w_ref, b_ref, o_ref):
        dinv = lax.rsqrt(dp_ref[0] + dp_ref[1] + 1.0)
        sg = jnp.concatenate([p_ref[0] + g_ref[0], p_ref[1] + g_ref[1]],
                             axis=1)
        a = sg * dinv
        h = jnp.dot(a, w_ref[...], preferred_element_type=jnp.float32)
        h = jnp.maximum(h + b_ref[...], 0.0)
        g = h * dinv
        o_ref[0] = g[:, :DH]
        o_ref[1] = g[:, DH:]

    return pl.pallas_call(
        body,
        grid=(N // C,),
        in_specs=[
            pl.BlockSpec((2, C, DH), lambda i: (0, i, 0)),
            pl.BlockSpec((2, C, DH), lambda i: (0, i, 0)),
            pl.BlockSpec((2, C, 1), lambda i: (0, i, 0)),
            pl.BlockSpec((D, D), lambda i: (0, 0)),
            pl.BlockSpec((1, D), lambda i: (0, 0)),
        ],
        out_specs=pl.BlockSpec((2, C, DH), lambda i: (0, i, 0)),
        out_shape=jax.ShapeDtypeStruct((2, N, DH), jnp.float32),
    )


def _tc_gcn_pool_head(N, D, C, G):
    nblk = N // C
    DH = D // 2

    def body(p_ref, g_ref, dp_ref, w_ref, b_ref, bt_ref, wo_ref, bo_ref,
             o_ref, pooled, cnt):
        i = pl.program_id(0)
        dinv = lax.rsqrt(dp_ref[0] + dp_ref[1] + 1.0)
        sg = jnp.concatenate([p_ref[0] + g_ref[0], p_ref[1] + g_ref[1]],
                             axis=1)
        a = sg * dinv
        h = jnp.dot(a, w_ref[...], preferred_element_type=jnp.float32)
        h = jnp.maximum(h + b_ref[...], 0.0)

        iota = lax.broadcasted_iota(jnp.int32, (C, G), 1)
        maskT = (bt_ref[...] == iota).astype(jnp.float32)
        pblk = lax.dot_general(maskT, h, (((0,), (0,)), ((), ())),
                               preferred_element_type=jnp.float32)
        ones = jnp.ones((C, 1), jnp.float32)
        cblk = lax.dot_general(maskT, ones, (((0,), (0,)), ((), ())),
                               preferred_element_type=jnp.float32)

        @pl.when(i == 0)
        def _():
            pooled[...] = pblk
            cnt[...] = cblk

        @pl.when(i > 0)
        def _():
            pooled[...] += pblk
            cnt[...] += cblk

        @pl.when(i == nblk - 1)
        def _():
            mean = pooled[...] / jnp.maximum(cnt[...], 1.0)
            o_ref[...] = jnp.dot(mean, wo_ref[...],
                                 preferred_element_type=jnp.float32) + bo_ref[...]

    return pl.pallas_call(
        body,
        grid=(nblk,),
        in_specs=[
            pl.BlockSpec((2, C, DH), lambda i: (0, i, 0)),
            pl.BlockSpec((2, C, DH), lambda i: (0, i, 0)),
            pl.BlockSpec((2, C, 1), lambda i: (0, i, 0)),
            pl.BlockSpec((D, D), lambda i: (0, 0)),
            pl.BlockSpec((1, D), lambda i: (0, 0)),
            pl.BlockSpec((C, 1), lambda i: (i, 0)),
            pl.BlockSpec((D, 1), lambda i: (0, 0)),
            pl.BlockSpec((1, 1), lambda i: (0, 0)),
        ],
        out_specs=pl.BlockSpec((G, 1), lambda i: (0, 0)),
        out_shape=jax.ShapeDtypeStruct((G, 1), jnp.float32),
        scratch_shapes=[
            pltpu.VMEM((G, D), jnp.float32),
            pltpu.VMEM((G, 1), jnp.float32),
        ],
    )


def kernel(x, edge_index, batch, W_in, b_in, W1, b1, W2, b2, W_out, b_out):
    N, D = x.shape
    E = edge_index.shape[1]
    G = 64
    C = 1000
    DH = D // 2
    NPAD = -(-(N + 1) // (_NS * _CH)) * (_NS * _CH)

    nch = -(-E // (_NS * _CH))
    nch += (-nch) % 4
    padn = _NS * nch * _CH - E
    src = edge_index[0]
    dst = edge_index[1]
    if padn:
        fill = jnp.arange(padn, dtype=jnp.int32)
        src = jnp.concatenate([src, fill % N])
        dst = jnp.concatenate([dst, N + fill % (NPAD - N)])
    src3 = src.reshape(_NS, nch, _CH)
    dst3 = dst.reshape(_NS, nch, _CH)
    nch_deg = nch // 2
    dst_deg = dst.reshape(_NC * _NS, nch_deg, _CH)

    degp = _sc_degree(nch_deg, NPAD)(dst_deg)
    dp = degp[:, :, None]

    g0 = _tc_lin_in(N, D, C)(x, W_in, b_in.reshape(1, D), dp)
    S = _sc_aggregate(nch, NPAD, DH)(src3, dst3, g0)
    g1 = _tc_gcn_mid(N, D, C)(S, g0, dp, W1, b1.reshape(1, D))
    S = _sc_aggregate(nch, NPAD, DH)(src3, dst3, g1)
    out = _tc_gcn_pool_head(N, D, C, G)(
        S, g1, dp, W2, b2.reshape(1, D), batch[:, None],
        W_out, b_out.reshape(1, 1))
    return out
